# initial kernel scaffold (unmeasured)
import jax
import jax.numpy as jnp
from jax import lax
from jax.experimental import pallas as pl
from jax.experimental.pallas import tpu as pltpu

N_DEV = 16
B, SQ, D = 1, 512, 1024
HQ, HKV, DH = 8, 2, 128
GROUP = HQ // HKV
SCALE = 0.08838834764831843
CHUNK = SQ // N_DEV


def kernel(x, Wq, Wo, K_ext, V_ext):
    skv = K_ext.shape[1]
    x2 = x.reshape(SQ, D)
    k2 = K_ext.reshape(skv, HKV * DH)
    v2 = V_ext.reshape(skv, HKV * DH)

    def body(x_ref, wq_ref, wo_ref, k_ref, v_ref, out_ref,
             o_scr, l_scr, co_ref, cl_ref, y_ref, ag_ref,
             o_send, o_recv, l_send, l_recv, g_send, g_recv):
        my = lax.axis_index("i")
        left = (my - 1) % N_DEV
        right = (my + 1) % N_DEV

        barrier = pltpu.get_barrier_semaphore()
        for nbr in (left, right):
            pl.semaphore_signal(barrier, inc=1, device_id=(nbr,),
                                device_id_type=pl.DeviceIdType.MESH)
        pl.semaphore_wait(barrier, 2)

        q = jnp.dot(x_ref[...], wq_ref[...],
                    preferred_element_type=jnp.float32)
        for h in range(HQ):
            g = h // GROUP
            qh = q[:, h * DH:(h + 1) * DH]
            kh = k_ref[:, g * DH:(g + 1) * DH]
            vh = v_ref[:, g * DH:(g + 1) * DH]
            s = lax.dot_general(qh, kh, (((1,), (1,)), ((), ())),
                                preferred_element_type=jnp.float32) * SCALE
            p = jnp.exp(s)
            l_scr[:, h:h + 1] = jnp.sum(p, axis=1, keepdims=True)
            o_scr[:, h * DH:(h + 1) * DH] = jnp.dot(
                p, vh, preferred_element_type=jnp.float32)

        c0 = ((my - 1) % N_DEV) * CHUNK
        co_ref[0] = o_scr[pl.ds(c0, CHUNK), :]
        cl_ref[0] = l_scr[pl.ds(c0, CHUNK), :]
        for s_ in range(N_DEV - 1):
            snd, rcv = s_ % 2, (s_ + 1) % 2
            rdo = pltpu.make_async_remote_copy(
                src_ref=co_ref.at[snd], dst_ref=co_ref.at[rcv],
                send_sem=o_send.at[snd], recv_sem=o_recv.at[rcv],
                device_id=(right,), device_id_type=pl.DeviceIdType.MESH)
            rdl = pltpu.make_async_remote_copy(
                src_ref=cl_ref.at[snd], dst_ref=cl_ref.at[rcv],
                send_sem=l_send.at[snd], recv_sem=l_recv.at[rcv],
                device_id=(right,), device_id_type=pl.DeviceIdType.MESH)
            rdo.start()
            rdl.start()
            rdo.wait()
            rdl.wait()
            c = ((my - s_ - 2) % N_DEV) * CHUNK
            co_ref[rcv] = co_ref[rcv] + o_scr[pl.ds(c, CHUNK), :]
            cl_ref[rcv] = cl_ref[rcv] + l_scr[pl.ds(c, CHUNK), :]

        fin = (N_DEV - 1) % 2
        for h in range(HQ):
            y_ref[:, h * DH:(h + 1) * DH] = (
                co_ref[fin][:, h * DH:(h + 1) * DH]
                / cl_ref[fin][:, h:h + 1])
        y = jnp.dot(y_ref[...], wo_ref[...],
                    preferred_element_type=jnp.float32)

        out_ref[pl.ds(my * CHUNK, CHUNK), :] = y
        ag_ref[0] = y
        for s_ in range(N_DEV - 1):
            snd, rcv = s_ % 2, (s_ + 1) % 2
            rd = pltpu.make_async_remote_copy(
                src_ref=ag_ref.at[snd], dst_ref=ag_ref.at[rcv],
                send_sem=g_send.at[snd], recv_sem=g_recv.at[rcv],
                device_id=(right,), device_id_type=pl.DeviceIdType.MESH)
            rd.start()
            rd.wait()
            c = ((my - s_ - 1) % N_DEV) * CHUNK
            out_ref[pl.ds(c, CHUNK), :] = ag_ref[rcv]

    out = pl.pallas_call(
        body,
        out_shape=jax.ShapeDtypeStruct((SQ, D), jnp.float32),
        in_specs=[pl.BlockSpec(memory_space=pltpu.VMEM)] * 5,
        out_specs=pl.BlockSpec(memory_space=pltpu.VMEM),
        scratch_shapes=[
            pltpu.VMEM((SQ, D), jnp.float32),
            pltpu.VMEM((SQ, HQ), jnp.float32),
            pltpu.VMEM((2, CHUNK, D), jnp.float32),
            pltpu.VMEM((2, CHUNK, HQ), jnp.float32),
            pltpu.VMEM((CHUNK, D), jnp.float32),
            pltpu.VMEM((2, CHUNK, D), jnp.float32),
            pltpu.SemaphoreType.DMA((2,)),
            pltpu.SemaphoreType.DMA((2,)),
            pltpu.SemaphoreType.DMA((2,)),
            pltpu.SemaphoreType.DMA((2,)),
            pltpu.SemaphoreType.DMA((2,)),
            pltpu.SemaphoreType.DMA((2,)),
        ],
        compiler_params=pltpu.CompilerParams(collective_id=0),
    )(x2, wq_w(Wq), Wo, k2, v2)
    return out.reshape(B, SQ, D)


def wq_w(Wq):
    return Wq


# baseline (device time: 123589 ns/iter reference)
import jax
import jax.numpy as jnp
from jax import lax
from jax.experimental import pallas as pl
from jax.experimental.pallas import tpu as pltpu

N_DEV = 16
B, SQ, D = 1, 512, 1024
HQ, HKV, DH = 8, 2, 128
GROUP = HQ // HKV
SCALE = 0.08838834764831843
CHUNK = SQ // N_DEV


def kernel(x, Wq, Wo, K_ext, V_ext):
    skv = K_ext.shape[1]
    x2 = x.reshape(SQ, D)
    k2 = K_ext.reshape(skv, HKV * DH)
    v2 = V_ext.reshape(skv, HKV * DH)

    def body(x_ref, wq_ref, wo_ref, k_ref, v_ref, out_ref,
             o_scr, l_scr, co_ref, cl_ref, y_ref, ag_ref,
             o_send, o_recv, l_send, l_recv, g_send, g_recv):
        my = lax.axis_index("i")
        left = (my - 1) % N_DEV
        right = (my + 1) % N_DEV

        barrier = pltpu.get_barrier_semaphore()
        for nbr in (left, right):
            pl.semaphore_signal(barrier, inc=1, device_id=(nbr,),
                                device_id_type=pl.DeviceIdType.MESH)
        pl.semaphore_wait(barrier, 2)

        q = jnp.dot(x_ref[...], wq_ref[...],
                    preferred_element_type=jnp.float32)
        for h in range(HQ):
            g = h // GROUP
            qh = q[:, h * DH:(h + 1) * DH]
            kh = k_ref[:, g * DH:(g + 1) * DH]
            vh = v_ref[:, g * DH:(g + 1) * DH]
            s = lax.dot_general(qh, kh, (((1,), (1,)), ((), ())),
                                preferred_element_type=jnp.float32) * SCALE
            p = jnp.exp(s)
            l_scr[:, h:h + 1] = jnp.sum(p, axis=1, keepdims=True)
            o_scr[:, h * DH:(h + 1) * DH] = jnp.dot(
                p, vh, preferred_element_type=jnp.float32)

        c0 = ((my - 1) % N_DEV) * CHUNK
        co_ref[0] = o_scr[pl.ds(c0, CHUNK), :]
        cl_ref[0] = l_scr[pl.ds(c0, CHUNK), :]
        for s_ in range(N_DEV - 1):
            snd, rcv = s_ % 2, (s_ + 1) % 2
            rdo = pltpu.make_async_remote_copy(
                src_ref=co_ref.at[snd], dst_ref=co_ref.at[rcv],
                send_sem=o_send.at[snd], recv_sem=o_recv.at[rcv],
                device_id=(right,), device_id_type=pl.DeviceIdType.MESH)
            rdl = pltpu.make_async_remote_copy(
                src_ref=cl_ref.at[snd], dst_ref=cl_ref.at[rcv],
                send_sem=l_send.at[snd], recv_sem=l_recv.at[rcv],
                device_id=(right,), device_id_type=pl.DeviceIdType.MESH)
            rdo.start()
            rdl.start()
            rdo.wait()
            rdl.wait()
            c = ((my - s_ - 2) % N_DEV) * CHUNK
            co_ref[rcv] = co_ref[rcv] + o_scr[pl.ds(c, CHUNK), :]
            cl_ref[rcv] = cl_ref[rcv] + l_scr[pl.ds(c, CHUNK), :]

        fin = (N_DEV - 1) % 2
        for h in range(HQ):
            y_ref[:, h * DH:(h + 1) * DH] = (
                co_ref[fin][:, h * DH:(h + 1) * DH]
                / cl_ref[fin][:, h:h + 1])
        y = jnp.dot(y_ref[...], wo_ref[...],
                    preferred_element_type=jnp.float32)

        out_ref[pl.ds(my * CHUNK, CHUNK), :] = y
        ag_ref[0] = y
        for s_ in range(N_DEV - 1):
            snd, rcv = s_ % 2, (s_ + 1) % 2
            rd = pltpu.make_async_remote_copy(
                src_ref=ag_ref.at[snd], dst_ref=ag_ref.at[rcv],
                send_sem=g_send.at[snd], recv_sem=g_recv.at[rcv],
                device_id=(right,), device_id_type=pl.DeviceIdType.MESH)
            rd.start()
            rd.wait()
            c = ((my - s_ - 1) % N_DEV) * CHUNK
            out_ref[pl.ds(c, CHUNK), :] = ag_ref[rcv]

    out = pl.pallas_call(
        body,
        out_shape=jax.ShapeDtypeStruct((SQ, D), jnp.float32),
        in_specs=[pl.BlockSpec(memory_space=pltpu.VMEM)] * 5,
        out_specs=pl.BlockSpec(memory_space=pltpu.VMEM),
        scratch_shapes=[
            pltpu.VMEM((SQ, D), jnp.float32),
            pltpu.VMEM((SQ, HQ), jnp.float32),
            pltpu.VMEM((2, CHUNK, D), jnp.float32),
            pltpu.VMEM((2, CHUNK, HQ), jnp.float32),
            pltpu.VMEM((CHUNK, D), jnp.float32),
            pltpu.VMEM((2, CHUNK, D), jnp.float32),
            pltpu.SemaphoreType.DMA((2,)),
            pltpu.SemaphoreType.DMA((2,)),
            pltpu.SemaphoreType.DMA((2,)),
            pltpu.SemaphoreType.DMA((2,)),
            pltpu.SemaphoreType.DMA((2,)),
            pltpu.SemaphoreType.DMA((2,)),
        ],
        compiler_params=pltpu.CompilerParams(collective_id=0),
    )(x2, Wq, Wo, k2, v2)
    return out.reshape(B, SQ, D)


# device time: 22410 ns/iter; 5.5149x vs baseline; 5.5149x over previous
import jax
import jax.numpy as jnp
from jax import lax
from jax.experimental import pallas as pl
from jax.experimental.pallas import tpu as pltpu

N_DEV = 16
B, SQ, D = 1, 512, 1024
HQ, HKV, DH = 8, 2, 128
GROUP = HQ // HKV
SCALE = 0.08838834764831843
CHUNK = SQ // N_DEV


def kernel(x, Wq, Wo, K_ext, V_ext):
    skv = K_ext.shape[1]
    x2 = x.reshape(SQ, D)
    k2 = K_ext.reshape(skv, HKV * DH)
    v2 = V_ext.reshape(skv, HKV * DH)

    def body(x_ref, wq_ref, wo_ref, k_ref, v_ref, out_ref, o_scr, l_scr, y_ref):
        q = jnp.dot(x_ref[...], wq_ref[...],
                    preferred_element_type=jnp.float32)
        for h in range(HQ):
            g = h // GROUP
            qh = q[:, h * DH:(h + 1) * DH]
            kh = k_ref[:, g * DH:(g + 1) * DH]
            vh = v_ref[:, g * DH:(g + 1) * DH]
            s = lax.dot_general(qh, kh, (((1,), (1,)), ((), ())),
                                preferred_element_type=jnp.float32) * SCALE
            p = jnp.exp(s)
            l_scr[:, h:h + 1] = jnp.sum(p, axis=1, keepdims=True)
            o_scr[:, h * DH:(h + 1) * DH] = jnp.dot(
                p, vh, preferred_element_type=jnp.float32)

        for h in range(HQ):
            y_ref[:, h * DH:(h + 1) * DH] = (
                o_scr[:, h * DH:(h + 1) * DH] / l_scr[:, h:h + 1])
        out_ref[...] = jnp.dot(y_ref[...], wo_ref[...],
                               preferred_element_type=jnp.float32)

    out = pl.pallas_call(
        body,
        out_shape=jax.ShapeDtypeStruct((SQ, D), jnp.float32),
        in_specs=[pl.BlockSpec(memory_space=pltpu.VMEM)] * 5,
        out_specs=pl.BlockSpec(memory_space=pltpu.VMEM),
        scratch_shapes=[
            pltpu.VMEM((SQ, D), jnp.float32),
            pltpu.VMEM((SQ, HQ), jnp.float32),
            pltpu.VMEM((SQ, D), jnp.float32),
        ],
    )(x2, Wq, Wo, k2, v2)
    return out.reshape(B, SQ, D)
